# baseline (speedup 1.0000x reference)
import functools

import jax
import jax.numpy as jnp
from jax import lax
from jax.experimental import pallas as pl
from jax.experimental.pallas import tpu as pltpu
from jax.experimental.pallas import tpu_sc as plsc

N = 10000
B = 2
M = 32 * 32 * 32
KH = 64
C = 128
OUT = 128
E = 320000
NTILES = 32
EPT = E // NTILES
ROWS = 400

HCH = 400


def _hist_body(dst_hbm, zeros_hbm, hist_out, dst0, dst1, hist_v,
               sem_h0, sem_h1):
    c = lax.axis_index("c")
    s = lax.axis_index("s")
    wid = c * 16 + s
    base = wid * EPT
    dstb = (dst0, dst1)
    sem = (sem_h0, sem_h1)

    def issue(i, b):
        pltpu.async_copy(dst_hbm.at[pl.ds(base + i * HCH, HCH)], dstb[b],
                         sem[b])

    def wait(b):
        pltpu.make_async_copy(dst_hbm.at[pl.ds(0, HCH)], dstb[b],
                              sem[b]).wait()

    issue(0, 0)
    issue(1, 1)
    pltpu.sync_copy(zeros_hbm, hist_v)
    ones = jnp.full((16,), 1.0, jnp.float32)
    nch = EPT // HCH

    def chunk(i2, _):
        for b in range(2):
            i = 2 * i2 + b
            wait(b)

            def vstep(j, _):
                d = dstb[b][pl.ds(j * 16, 16)]
                plsc.addupdate_scatter(hist_v, [d], ones)
                return 0

            lax.fori_loop(0, HCH // 16, vstep, 0, unroll=True)

            @pl.when(i + 2 <= nch - 1)
            def _():
                issue(i + 2, b)
        return 0

    lax.fori_loop(0, nch // 2, chunk, 0)
    wait(0)

    def vstep_last(j, _):
        d = dst0[pl.ds(j * 16, 16)]
        plsc.addupdate_scatter(hist_v, [d], ones)
        return 0

    lax.fori_loop(0, HCH // 16, vstep_last, 0, unroll=True)
    pltpu.sync_copy(hist_v, hist_out.at[wid])


def _hist(dst, zeros1):
    mesh = plsc.VectorSubcoreMesh(core_axis_name="c", subcore_axis_name="s")
    f = pl.kernel(
        _hist_body,
        out_type=jax.ShapeDtypeStruct((NTILES, B * M), jnp.float32),
        mesh=mesh,
        scratch_types=[
            pltpu.VMEM((HCH,), jnp.int32),
            pltpu.VMEM((HCH,), jnp.int32),
            pltpu.VMEM((B * M,), jnp.float32),
            pltpu.SemaphoreType.DMA,
            pltpu.SemaphoreType.DMA,
        ],
        compiler_params=pltpu.CompilerParams(needs_layout_passes=False),
    )
    return f(dst, zeros1)


ECH = 80
NCH = EPT // ECH
_G2 = 1.5957691216057308
_GA = 0.044715


def _edge_chunk_compute(src_b, u_rows, v_rows, a_sh):
    def row(r, _):
        wv = u_rows[r, pl.ds(KH, 16)]
        for c4 in range(4):
            sl = pl.ds(c4 * 16, 16)
            z = u_rows[r, sl] + v_rows[r, sl]
            z2 = z * z
            y = _G2 * (z + _GA * z2 * z)
            g = z / (1.0 + jnp.exp(-y))
            u_rows[r, sl] = g * wv
        return 0

    lax.fori_loop(0, ECH, row, 0, unroll=4)
    pltpu.sync_copy(u_rows, a_sh.at[src_b], add=True)


def _edge_body(src_hbm, dst_hbm, u2_hbm, v2_hbm, zerosa_hbm,
               a_out,
               a_sh,
               src0, src1, dst0, dst1, u0, u1, v0, v1,
               sem_i0, sem_i1, sem_u0, sem_u1, sem_v0, sem_v1):
    c = lax.axis_index("c")
    s = lax.axis_index("s")
    wid = c * 16 + s
    base = wid * EPT

    @pl.when(s < 10)
    def _():
        pltpu.sync_copy(zerosa_hbm.at[pl.ds(s * 1000, 1000)],
                        a_sh.at[pl.ds(s * 1000, 1000)])

    plsc.subcore_barrier()

    srcb = (src0, src1)
    dstb = (dst0, dst1)
    ur = (u0, u1)
    vr = (v0, v1)
    sem_i = (sem_i0, sem_i1)
    sem_u = (sem_u0, sem_u1)
    sem_v = (sem_v0, sem_v1)
    def issue_idx(ch, b):
        off = base + ch * ECH
        pltpu.async_copy(src_hbm.at[pl.ds(off, ECH)], srcb[b], sem_i[b])
        pltpu.async_copy(dst_hbm.at[pl.ds(off, ECH)], dstb[b], sem_i[b])

    def wait_idx(b):
        pltpu.make_async_copy(src_hbm.at[pl.ds(0, ECH)], srcb[b],
                              sem_i[b]).wait()
        pltpu.make_async_copy(dst_hbm.at[pl.ds(0, ECH)], dstb[b],
                              sem_i[b]).wait()

    def issue_gathers(b):
        pltpu.async_copy(u2_hbm.at[dstb[b]], ur[b], sem_u[b])
        pltpu.async_copy(v2_hbm.at[srcb[b]], vr[b], sem_v[b])

    def wait_gathers(b):
        pltpu.make_async_copy(u2_hbm.at[dstb[b]], ur[b], sem_u[b]).wait()
        pltpu.make_async_copy(v2_hbm.at[srcb[b]], vr[b], sem_v[b]).wait()

    pltpu.sync_copy(src_hbm.at[pl.ds(base, ECH)], srcb[0])
    pltpu.sync_copy(dst_hbm.at[pl.ds(base, ECH)], dstb[0])
    issue_gathers(0)
    issue_idx(1, 1)

    def pair(j, _):
        for b in range(2):
            ch = 2 * j + b
            nb = 1 - b
            wait_idx(nb)
            issue_gathers(nb)
            wait_gathers(b)
            _edge_chunk_compute(srcb[b], ur[b], vr[b], a_sh)

            @pl.when(ch + 2 <= NCH - 1)
            def _():
                issue_idx(ch + 2, b)
        return 0

    lax.fori_loop(0, (NCH - 1) // 2, pair, 0)
    wait_gathers(0)
    _edge_chunk_compute(srcb[0], ur[0], vr[0], a_sh)

    plsc.subcore_barrier()

    @pl.when(s == 0)
    def _():
        pltpu.sync_copy(a_sh, a_out.at[c])


def _edge(src, dst, u2, v2, zerosa):
    mesh = plsc.VectorSubcoreMesh(core_axis_name="c", subcore_axis_name="s")
    f = pl.kernel(
        _edge_body,
        out_type=jax.ShapeDtypeStruct((2, N, C), jnp.float32),
        mesh=mesh,
        scratch_types=[
            pltpu.VMEM_SHARED((N, C), jnp.float32),
            pltpu.VMEM((ECH,), jnp.int32),
            pltpu.VMEM((ECH,), jnp.int32),
            pltpu.VMEM((ECH,), jnp.int32),
            pltpu.VMEM((ECH,), jnp.int32),
            pltpu.VMEM((ECH, C), jnp.float32),
            pltpu.VMEM((ECH, C), jnp.float32),
            pltpu.VMEM((ECH, C), jnp.float32),
            pltpu.VMEM((ECH, C), jnp.float32),
            pltpu.SemaphoreType.DMA,
            pltpu.SemaphoreType.DMA,
            pltpu.SemaphoreType.DMA,
            pltpu.SemaphoreType.DMA,
            pltpu.SemaphoreType.DMA,
            pltpu.SemaphoreType.DMA,
        ],
        compiler_params=pltpu.CompilerParams(needs_layout_passes=False),
    )
    return f(src, dst, u2, v2, zerosa)


def _w_body(hist_ref, w_ref):
    cnt = jnp.sum(hist_ref[...], axis=0)
    w = 1.0 / jnp.maximum(cnt, 1.0)
    w_ref[...] = w.reshape(16, 128)


def _w_from_hist(hist_p):
    w2 = pl.pallas_call(
        _w_body,
        grid=(32,),
        in_specs=[pl.BlockSpec((NTILES, 2048), lambda i: (0, i))],
        out_specs=pl.BlockSpec((16, 128), lambda i: (i, 0)),
        out_shape=jax.ShapeDtypeStruct((512, 128), jnp.float32),
    )(hist_p)
    return w2.reshape(B * M, 1)


def _u2_body(lat8_ref, k8_ref, b_ref, wcol_ref, o_ref):
    u = jnp.dot(lat8_ref[...], k8_ref[...],
                preferred_element_type=jnp.float32) + b_ref[...]
    wrep = jnp.broadcast_to(wcol_ref[...], (wcol_ref.shape[0], 16))
    pad = jnp.zeros((u.shape[0], C - KH - 16), jnp.float32)
    o_ref[...] = jnp.concatenate([u, wrep, pad], axis=1)


def _build_u2(lat8, kw1_8, kb1, wcol):
    rows = 1024
    return pl.pallas_call(
        _u2_body,
        grid=(2, M // rows),
        in_specs=[
            pl.BlockSpec((rows, 8), lambda i, j: (j, 0)),
            pl.BlockSpec((8, KH), lambda i, j: (0, 0)),
            pl.BlockSpec((1, KH), lambda i, j: (0, 0)),
            pl.BlockSpec((rows, 1), lambda i, j: (i * (M // rows) + j, 0)),
        ],
        out_specs=pl.BlockSpec((rows, C), lambda i, j: (i * (M // rows) + j, 0)),
        out_shape=jax.ShapeDtypeStruct((B * M, C), jnp.float32),
    )(lat8, kw1_8, kb1, wcol)


def _v2_body(pos8_ref, k8_ref, o_ref):
    v = -jnp.dot(pos8_ref[...], k8_ref[...], preferred_element_type=jnp.float32)
    pad = jnp.zeros((v.shape[0], C - KH), jnp.float32)
    o_ref[...] = jnp.concatenate([v, pad], axis=1)


def _build_v2(pos8, kw1_8):
    return pl.pallas_call(
        _v2_body,
        grid=(N // ROWS,),
        in_specs=[
            pl.BlockSpec((ROWS, 8), lambda i: (i, 0)),
            pl.BlockSpec((8, KH), lambda i: (0, 0)),
        ],
        out_specs=pl.BlockSpec((ROWS, C), lambda i: (i, 0)),
        out_shape=jax.ShapeDtypeStruct((N, C), jnp.float32),
    )(pos8, kw1_8)


def _combine_body(x_ref, a_ref, bidx_ref, wl_ref, bl_ref, kw2_ref,
                  kb2_ref, y_ref, acc_ref):
    i = pl.program_id(0)

    @pl.when(i == 0)
    def _():
        acc_ref[...] = jnp.zeros_like(acc_ref)

    lifted = jnp.dot(x_ref[...], wl_ref[...],
                     preferred_element_type=jnp.float32) + bl_ref[...]
    a = a_ref[0] + a_ref[1]
    q = jnp.dot(a[:, :KH], kw2_ref[...], preferred_element_type=jnp.float32)
    cw = a[:, KH:KH + 1]
    r = q + cw * kb2_ref[...]
    prod = r * lifted
    bidx = bidx_ref[...]
    for b in range(B):
        mask = (bidx == b).astype(jnp.float32)
        acc_ref[b, :] += jnp.sum(prod * mask, axis=0)

    @pl.when(i == pl.num_programs(0) - 1)
    def _():
        y_ref[...] = acc_ref[...]


def _expand_body(y_ref, wout_ref, bout_ref, bidx_ref, out_ref):
    pooled = y_ref[...] * (1.0 / M)
    rows = jnp.dot(pooled, wout_ref[...],
                   preferred_element_type=jnp.float32) + bout_ref[...]
    bidx = bidx_ref[...]
    out_ref[...] = jnp.where(bidx == 0, rows[0, :][None, :], rows[1, :][None, :])


def kernel(x, pos, W_lift, b_lift, K_W1, K_b1, K_W2, K_b2, W_out, b_out,
           latent_tokens, edge_index, batch_idx):
    dst = edge_index[1]
    lat8 = jnp.pad(latent_tokens, ((0, 0), (0, 5)))
    pos8 = jnp.pad(pos, ((0, 0), (0, 5)))
    kw1_8 = jnp.pad(K_W1, ((0, 5), (0, 0)))
    zeros1 = jnp.zeros((B * M,), jnp.float32)
    zerosa = jnp.zeros((N, C), jnp.float32)

    hist_p = _hist(dst, zeros1)
    wcol = _w_from_hist(hist_p)
    u2 = _build_u2(lat8, kw1_8, K_b1.reshape(1, KH), wcol)
    v2 = _build_v2(pos8, kw1_8)
    a_p = _edge(edge_index[0], dst, u2, v2, zerosa)

    bidx2d = batch_idx.reshape(N, 1)
    grid = N // ROWS
    y = pl.pallas_call(
        _combine_body,
        grid=(grid,),
        in_specs=[
            pl.BlockSpec((ROWS, 128), lambda i: (i, 0)),
            pl.BlockSpec((2, ROWS, C), lambda i: (0, i, 0)),
            pl.BlockSpec((ROWS, 1), lambda i: (i, 0)),
            pl.BlockSpec((128, C), lambda i: (0, 0)),
            pl.BlockSpec((1, C), lambda i: (0, 0)),
            pl.BlockSpec((KH, C), lambda i: (0, 0)),
            pl.BlockSpec((1, C), lambda i: (0, 0)),
        ],
        out_specs=pl.BlockSpec((B, C), lambda i: (0, 0)),
        out_shape=jax.ShapeDtypeStruct((B, C), jnp.float32),
        scratch_shapes=[pltpu.VMEM((B, C), jnp.float32)],
    )(x, a_p, bidx2d, W_lift, b_lift.reshape(1, C), K_W2, K_b2.reshape(1, C))

    out = pl.pallas_call(
        _expand_body,
        grid=(grid,),
        in_specs=[
            pl.BlockSpec((B, C), lambda i: (0, 0)),
            pl.BlockSpec((C, OUT), lambda i: (0, 0)),
            pl.BlockSpec((1, OUT), lambda i: (0, 0)),
            pl.BlockSpec((ROWS, 1), lambda i: (i, 0)),
        ],
        out_specs=pl.BlockSpec((ROWS, OUT), lambda i: (i, 0)),
        out_shape=jax.ShapeDtypeStruct((N, OUT), jnp.float32),
    )(y, W_out, b_out.reshape(1, OUT), bidx2d)
    return out


# speedup vs baseline: 2.3225x; 2.3225x over previous
import functools

import jax
import jax.numpy as jnp
from jax import lax
from jax.experimental import pallas as pl
from jax.experimental.pallas import tpu as pltpu
from jax.experimental.pallas import tpu_sc as plsc

N = 10000
B = 2
M = 32 * 32 * 32
KH = 64
C = 128
OUT = 128
E = 320000
NTILES = 32
EPT = E // NTILES
ROWS = 400

HCH = 400


def _hist_body(dst_hbm, zeros_hbm, hist_out, dst0, dst1, hist_v,
               sem_h0, sem_h1):
    c = lax.axis_index("c")
    s = lax.axis_index("s")
    wid = c * 16 + s
    base = wid * EPT
    dstb = (dst0, dst1)
    sem = (sem_h0, sem_h1)

    def issue(i, b):
        pltpu.async_copy(dst_hbm.at[pl.ds(base + i * HCH, HCH)], dstb[b],
                         sem[b])

    def wait(b):
        pltpu.make_async_copy(dst_hbm.at[pl.ds(0, HCH)], dstb[b],
                              sem[b]).wait()

    issue(0, 0)
    issue(1, 1)
    pltpu.sync_copy(zeros_hbm, hist_v)
    ones = jnp.full((16,), 1.0, jnp.float32)
    nch = EPT // HCH

    def chunk(i2, _):
        for b in range(2):
            i = 2 * i2 + b
            wait(b)

            def vstep(j, _):
                d = dstb[b][pl.ds(j * 16, 16)]
                plsc.addupdate_scatter(hist_v, [d], ones)
                return 0

            lax.fori_loop(0, HCH // 16, vstep, 0, unroll=True)

            @pl.when(i + 2 <= nch - 1)
            def _():
                issue(i + 2, b)
        return 0

    lax.fori_loop(0, nch // 2, chunk, 0)
    wait(0)

    def vstep_last(j, _):
        d = dst0[pl.ds(j * 16, 16)]
        plsc.addupdate_scatter(hist_v, [d], ones)
        return 0

    lax.fori_loop(0, HCH // 16, vstep_last, 0, unroll=True)
    pltpu.sync_copy(hist_v, hist_out.at[wid])


def _hist(dst, zeros1):
    mesh = plsc.VectorSubcoreMesh(core_axis_name="c", subcore_axis_name="s")
    f = pl.kernel(
        _hist_body,
        out_type=jax.ShapeDtypeStruct((NTILES, B * M), jnp.float32),
        mesh=mesh,
        scratch_types=[
            pltpu.VMEM((HCH,), jnp.int32),
            pltpu.VMEM((HCH,), jnp.int32),
            pltpu.VMEM((B * M,), jnp.float32),
            pltpu.SemaphoreType.DMA,
            pltpu.SemaphoreType.DMA,
        ],
        compiler_params=pltpu.CompilerParams(needs_layout_passes=False),
    )
    return f(dst, zeros1)


ECH = 80
NCH = EPT // ECH
_G2 = 1.5957691216057308
_GA = 0.044715


def _edge_chunk_compute(src_b, u_rows, v_rows, a_sh):
    def row(r, _):
        wv = u_rows[r, pl.ds(KH, 16)]
        for c4 in range(4):
            sl = pl.ds(c4 * 16, 16)
            z = u_rows[r, sl] + v_rows[r, sl]
            z2 = z * z
            y = _G2 * (z + _GA * z2 * z)
            g = z / (1.0 + jnp.exp(-y))
            u_rows[r, sl] = g * wv
        return 0

    lax.fori_loop(0, ECH, row, 0)
    pltpu.sync_copy(u_rows, a_sh.at[src_b], add=True)


def _edge_body(src_hbm, dst_hbm, u2_hbm, v2_hbm, zerosa_hbm,
               a_out,
               a_sh,
               src0, src1, dst0, dst1, u0, u1, v0, v1,
               sem_i0, sem_i1, sem_u0, sem_u1, sem_v0, sem_v1):
    c = lax.axis_index("c")
    s = lax.axis_index("s")
    wid = c * 16 + s
    base = wid * EPT

    @pl.when(s < 10)
    def _():
        pltpu.sync_copy(zerosa_hbm.at[pl.ds(s * 1000, 1000)],
                        a_sh.at[pl.ds(s * 1000, 1000)])

    plsc.subcore_barrier()

    srcb = (src0, src1)
    dstb = (dst0, dst1)
    ur = (u0, u1)
    vr = (v0, v1)
    sem_i = (sem_i0, sem_i1)
    sem_u = (sem_u0, sem_u1)
    sem_v = (sem_v0, sem_v1)
    def issue_idx(ch, b):
        off = base + ch * ECH
        pltpu.async_copy(src_hbm.at[pl.ds(off, ECH)], srcb[b], sem_i[b])
        pltpu.async_copy(dst_hbm.at[pl.ds(off, ECH)], dstb[b], sem_i[b])

    def wait_idx(b):
        pltpu.make_async_copy(src_hbm.at[pl.ds(0, ECH)], srcb[b],
                              sem_i[b]).wait()
        pltpu.make_async_copy(dst_hbm.at[pl.ds(0, ECH)], dstb[b],
                              sem_i[b]).wait()

    def issue_gathers(b):
        pltpu.async_copy(u2_hbm.at[dstb[b]], ur[b], sem_u[b])
        pltpu.async_copy(v2_hbm.at[srcb[b]], vr[b], sem_v[b])

    def wait_gathers(b):
        pltpu.make_async_copy(u2_hbm.at[dstb[b]], ur[b], sem_u[b]).wait()
        pltpu.make_async_copy(v2_hbm.at[srcb[b]], vr[b], sem_v[b]).wait()

    pltpu.sync_copy(src_hbm.at[pl.ds(base, ECH)], srcb[0])
    pltpu.sync_copy(dst_hbm.at[pl.ds(base, ECH)], dstb[0])
    issue_gathers(0)
    issue_idx(1, 1)

    def pair(j, _):
        for b in range(2):
            ch = 2 * j + b
            nb = 1 - b
            wait_idx(nb)
            issue_gathers(nb)
            wait_gathers(b)
            _edge_chunk_compute(srcb[b], ur[b], vr[b], a_sh)

            @pl.when(ch + 2 <= NCH - 1)
            def _():
                issue_idx(ch + 2, b)
        return 0

    lax.fori_loop(0, (NCH - 1) // 2, pair, 0)
    wait_gathers(0)
    _edge_chunk_compute(srcb[0], ur[0], vr[0], a_sh)

    plsc.subcore_barrier()

    @pl.when(s == 0)
    def _():
        pltpu.sync_copy(a_sh, a_out.at[c])


def _edge(src, dst, u2, v2, zerosa):
    mesh = plsc.VectorSubcoreMesh(core_axis_name="c", subcore_axis_name="s")
    f = pl.kernel(
        _edge_body,
        out_type=jax.ShapeDtypeStruct((2, N, C), jnp.float32),
        mesh=mesh,
        scratch_types=[
            pltpu.VMEM_SHARED((N, C), jnp.float32),
            pltpu.VMEM((ECH,), jnp.int32),
            pltpu.VMEM((ECH,), jnp.int32),
            pltpu.VMEM((ECH,), jnp.int32),
            pltpu.VMEM((ECH,), jnp.int32),
            pltpu.VMEM((ECH, C), jnp.float32),
            pltpu.VMEM((ECH, C), jnp.float32),
            pltpu.VMEM((ECH, C), jnp.float32),
            pltpu.VMEM((ECH, C), jnp.float32),
            pltpu.SemaphoreType.DMA,
            pltpu.SemaphoreType.DMA,
            pltpu.SemaphoreType.DMA,
            pltpu.SemaphoreType.DMA,
            pltpu.SemaphoreType.DMA,
            pltpu.SemaphoreType.DMA,
        ],
        compiler_params=pltpu.CompilerParams(needs_layout_passes=False),
    )
    return f(src, dst, u2, v2, zerosa)


def _w_body(hist_ref, w_ref):
    cnt = jnp.sum(hist_ref[...], axis=0)
    w = 1.0 / jnp.maximum(cnt, 1.0)
    w_ref[...] = w.reshape(16, 128)


def _w_from_hist(hist_p):
    w2 = pl.pallas_call(
        _w_body,
        grid=(32,),
        in_specs=[pl.BlockSpec((NTILES, 2048), lambda i: (0, i))],
        out_specs=pl.BlockSpec((16, 128), lambda i: (i, 0)),
        out_shape=jax.ShapeDtypeStruct((512, 128), jnp.float32),
    )(hist_p)
    return w2.reshape(B * M, 1)


def _u2_body(lat8_ref, k8_ref, b_ref, wcol_ref, o_ref):
    u = jnp.dot(lat8_ref[...], k8_ref[...],
                preferred_element_type=jnp.float32) + b_ref[...]
    wrep = jnp.broadcast_to(wcol_ref[...], (wcol_ref.shape[0], 16))
    pad = jnp.zeros((u.shape[0], C - KH - 16), jnp.float32)
    o_ref[...] = jnp.concatenate([u, wrep, pad], axis=1)


def _build_u2(lat8, kw1_8, kb1, wcol):
    rows = 1024
    return pl.pallas_call(
        _u2_body,
        grid=(2, M // rows),
        in_specs=[
            pl.BlockSpec((rows, 8), lambda i, j: (j, 0)),
            pl.BlockSpec((8, KH), lambda i, j: (0, 0)),
            pl.BlockSpec((1, KH), lambda i, j: (0, 0)),
            pl.BlockSpec((rows, 1), lambda i, j: (i * (M // rows) + j, 0)),
        ],
        out_specs=pl.BlockSpec((rows, C), lambda i, j: (i * (M // rows) + j, 0)),
        out_shape=jax.ShapeDtypeStruct((B * M, C), jnp.float32),
    )(lat8, kw1_8, kb1, wcol)


def _v2_body(pos8_ref, k8_ref, o_ref):
    v = -jnp.dot(pos8_ref[...], k8_ref[...], preferred_element_type=jnp.float32)
    pad = jnp.zeros((v.shape[0], C - KH), jnp.float32)
    o_ref[...] = jnp.concatenate([v, pad], axis=1)


def _build_v2(pos8, kw1_8):
    return pl.pallas_call(
        _v2_body,
        grid=(N // ROWS,),
        in_specs=[
            pl.BlockSpec((ROWS, 8), lambda i: (i, 0)),
            pl.BlockSpec((8, KH), lambda i: (0, 0)),
        ],
        out_specs=pl.BlockSpec((ROWS, C), lambda i: (i, 0)),
        out_shape=jax.ShapeDtypeStruct((N, C), jnp.float32),
    )(pos8, kw1_8)


def _combine_body(x_ref, a_ref, bidx_ref, wl_ref, bl_ref, kw2_ref,
                  kb2_ref, y_ref, acc_ref):
    i = pl.program_id(0)

    @pl.when(i == 0)
    def _():
        acc_ref[...] = jnp.zeros_like(acc_ref)

    lifted = jnp.dot(x_ref[...], wl_ref[...],
                     preferred_element_type=jnp.float32) + bl_ref[...]
    a = a_ref[0] + a_ref[1]
    q = jnp.dot(a[:, :KH], kw2_ref[...], preferred_element_type=jnp.float32)
    cw = a[:, KH:KH + 1]
    r = q + cw * kb2_ref[...]
    prod = r * lifted
    bidx = bidx_ref[...]
    for b in range(B):
        mask = (bidx == b).astype(jnp.float32)
        acc_ref[b, :] += jnp.sum(prod * mask, axis=0)

    @pl.when(i == pl.num_programs(0) - 1)
    def _():
        y_ref[...] = acc_ref[...]


def _expand_body(y_ref, wout_ref, bout_ref, bidx_ref, out_ref):
    pooled = y_ref[...] * (1.0 / M)
    rows = jnp.dot(pooled, wout_ref[...],
                   preferred_element_type=jnp.float32) + bout_ref[...]
    bidx = bidx_ref[...]
    out_ref[...] = jnp.where(bidx == 0, rows[0, :][None, :], rows[1, :][None, :])


def kernel(x, pos, W_lift, b_lift, K_W1, K_b1, K_W2, K_b2, W_out, b_out,
           latent_tokens, edge_index, batch_idx):
    dst = edge_index[1]
    lat8 = jnp.pad(latent_tokens, ((0, 0), (0, 5)))
    pos8 = jnp.pad(pos, ((0, 0), (0, 5)))
    kw1_8 = jnp.pad(K_W1, ((0, 5), (0, 0)))
    zeros1 = jnp.zeros((B * M,), jnp.float32)
    zerosa = jnp.zeros((N, C), jnp.float32)

    hist_p = _hist(dst, zeros1)
    wcol = _w_from_hist(hist_p)
    u2 = _build_u2(lat8, kw1_8, K_b1.reshape(1, KH), wcol)
    v2 = _build_v2(pos8, kw1_8)
    a_p = _edge(edge_index[0], dst, u2, v2, zerosa)

    bidx2d = batch_idx.reshape(N, 1)
    grid = N // ROWS
    y = pl.pallas_call(
        _combine_body,
        grid=(grid,),
        in_specs=[
            pl.BlockSpec((ROWS, 128), lambda i: (i, 0)),
            pl.BlockSpec((2, ROWS, C), lambda i: (0, i, 0)),
            pl.BlockSpec((ROWS, 1), lambda i: (i, 0)),
            pl.BlockSpec((128, C), lambda i: (0, 0)),
            pl.BlockSpec((1, C), lambda i: (0, 0)),
            pl.BlockSpec((KH, C), lambda i: (0, 0)),
            pl.BlockSpec((1, C), lambda i: (0, 0)),
        ],
        out_specs=pl.BlockSpec((B, C), lambda i: (0, 0)),
        out_shape=jax.ShapeDtypeStruct((B, C), jnp.float32),
        scratch_shapes=[pltpu.VMEM((B, C), jnp.float32)],
    )(x, a_p, bidx2d, W_lift, b_lift.reshape(1, C), K_W2, K_b2.reshape(1, C))

    out = pl.pallas_call(
        _expand_body,
        grid=(grid,),
        in_specs=[
            pl.BlockSpec((B, C), lambda i: (0, 0)),
            pl.BlockSpec((C, OUT), lambda i: (0, 0)),
            pl.BlockSpec((1, OUT), lambda i: (0, 0)),
            pl.BlockSpec((ROWS, 1), lambda i: (i, 0)),
        ],
        out_specs=pl.BlockSpec((ROWS, OUT), lambda i: (i, 0)),
        out_shape=jax.ShapeDtypeStruct((N, OUT), jnp.float32),
    )(y, W_out, b_out.reshape(1, OUT), bidx2d)
    return out


# speedup vs baseline: 2.3787x; 1.0242x over previous
import functools

import jax
import jax.numpy as jnp
from jax import lax
from jax.experimental import pallas as pl
from jax.experimental.pallas import tpu as pltpu
from jax.experimental.pallas import tpu_sc as plsc

N = 10000
B = 2
M = 32 * 32 * 32
KH = 64
C = 128
OUT = 128
E = 320000
NTILES = 32
EPT = E // NTILES
ROWS = 400

HCH = 400


def _hist_body(dst_hbm, zeros_hbm, hist_out, dst0, dst1, hist_v,
               sem_h0, sem_h1):
    c = lax.axis_index("c")
    s = lax.axis_index("s")
    wid = c * 16 + s
    base = wid * EPT
    dstb = (dst0, dst1)
    sem = (sem_h0, sem_h1)

    def issue(i, b):
        pltpu.async_copy(dst_hbm.at[pl.ds(base + i * HCH, HCH)], dstb[b],
                         sem[b])

    def wait(b):
        pltpu.make_async_copy(dst_hbm.at[pl.ds(0, HCH)], dstb[b],
                              sem[b]).wait()

    issue(0, 0)
    issue(1, 1)
    pltpu.sync_copy(zeros_hbm, hist_v)
    ones = jnp.full((16,), 1.0, jnp.float32)
    nch = EPT // HCH

    def chunk(i2, _):
        for b in range(2):
            i = 2 * i2 + b
            wait(b)

            def vstep(j, _):
                d = dstb[b][pl.ds(j * 16, 16)]
                plsc.addupdate_scatter(hist_v, [d], ones)
                return 0

            lax.fori_loop(0, HCH // 16, vstep, 0, unroll=True)

            @pl.when(i + 2 <= nch - 1)
            def _():
                issue(i + 2, b)
        return 0

    lax.fori_loop(0, nch // 2, chunk, 0)
    wait(0)

    def vstep_last(j, _):
        d = dst0[pl.ds(j * 16, 16)]
        plsc.addupdate_scatter(hist_v, [d], ones)
        return 0

    lax.fori_loop(0, HCH // 16, vstep_last, 0, unroll=True)
    pltpu.sync_copy(hist_v, hist_out.at[wid])


def _hist(dst, zeros1):
    mesh = plsc.VectorSubcoreMesh(core_axis_name="c", subcore_axis_name="s")
    f = pl.kernel(
        _hist_body,
        out_type=jax.ShapeDtypeStruct((NTILES, B * M), jnp.float32),
        mesh=mesh,
        scratch_types=[
            pltpu.VMEM((HCH,), jnp.int32),
            pltpu.VMEM((HCH,), jnp.int32),
            pltpu.VMEM((B * M,), jnp.float32),
            pltpu.SemaphoreType.DMA,
            pltpu.SemaphoreType.DMA,
        ],
        compiler_params=pltpu.CompilerParams(needs_layout_passes=False),
    )
    return f(dst, zeros1)


ECH = 96
NCH = EPT // ECH
TAIL = EPT - NCH * ECH
_G2 = 1.5957691216057308
_GA = 0.044715


def _edge_chunk_compute(adj_b, u_rows, v_rows, a_sh, n_rows):
    def row(r, _):
        wv = u_rows[r, pl.ds(KH, 16)]
        for c4 in range(4):
            sl = pl.ds(c4 * 16, 16)
            z = u_rows[r, sl] + v_rows[r, sl]
            z2 = z * z
            y = _G2 * (z + _GA * z2 * z)
            g = z / (1.0 + jnp.exp(-y))
            u_rows[r, sl] = g * wv
        return 0

    lax.fori_loop(0, n_rows, row, 0)
    pltpu.sync_copy(u_rows, a_sh.at[adj_b], add=True)


def _edge_body(src_hbm, dst_hbm, u2_hbm, v2_hbm, zerosa_hbm,
               a_out,
               a_sh,
               src0, src1, dst0, dst1, adj0, adj1, u0, u1, v0, v1,
               src_t, dst_t, adj_t, u_t, v_t,
               sem_i0, sem_i1, sem_u0, sem_u1, sem_v0, sem_v1):
    c = lax.axis_index("c")
    s = lax.axis_index("s")
    wid = c * 16 + s
    base = wid * EPT
    coff = c * (N // 2)

    @pl.when(s < 5)
    def _():
        pltpu.sync_copy(zerosa_hbm.at[pl.ds(s * 1000, 1000)],
                        a_sh.at[pl.ds(s * 1000, 1000)])

    plsc.subcore_barrier()

    srcb = (src0, src1)
    dstb = (dst0, dst1)
    adjb = (adj0, adj1)
    ur = (u0, u1)
    vr = (v0, v1)

    def adj(b, n16):
        def vstep(j, _):
            sl = pl.ds(j * 16, 16)
            adjb[b][sl] = srcb[b][sl] - coff
            return 0
        lax.fori_loop(0, n16, vstep, 0, unroll=True)
    sem_i = (sem_i0, sem_i1)
    sem_u = (sem_u0, sem_u1)
    sem_v = (sem_v0, sem_v1)
    def issue_idx(ch, b):
        off = base + ch * ECH
        pltpu.async_copy(src_hbm.at[pl.ds(off, ECH)], srcb[b], sem_i[b])
        pltpu.async_copy(dst_hbm.at[pl.ds(off, ECH)], dstb[b], sem_i[b])

    def wait_idx(b):
        pltpu.make_async_copy(src_hbm.at[pl.ds(0, ECH)], srcb[b],
                              sem_i[b]).wait()
        pltpu.make_async_copy(dst_hbm.at[pl.ds(0, ECH)], dstb[b],
                              sem_i[b]).wait()

    def issue_gathers(b):
        pltpu.async_copy(u2_hbm.at[dstb[b]], ur[b], sem_u[b])
        pltpu.async_copy(v2_hbm.at[srcb[b]], vr[b], sem_v[b])

    def wait_gathers(b):
        pltpu.make_async_copy(u2_hbm.at[dstb[b]], ur[b], sem_u[b]).wait()
        pltpu.make_async_copy(v2_hbm.at[srcb[b]], vr[b], sem_v[b]).wait()

    pltpu.sync_copy(src_hbm.at[pl.ds(base, ECH)], srcb[0])
    pltpu.sync_copy(dst_hbm.at[pl.ds(base, ECH)], dstb[0])
    issue_gathers(0)
    adj(0, ECH // 16)
    issue_idx(1, 1)

    def pair(j, _):
        for b in range(2):
            ch = 2 * j + b
            nb = 1 - b
            @pl.when(ch + 1 <= NCH - 1)
            def _():
                wait_idx(nb)
                issue_gathers(nb)
                adj(nb, ECH // 16)

            wait_gathers(b)
            _edge_chunk_compute(adjb[b], ur[b], vr[b], a_sh, ECH)

            @pl.when(ch + 2 <= NCH - 1)
            def _():
                issue_idx(ch + 2, b)
        return 0

    lax.fori_loop(0, NCH // 2, pair, 0)
    toff = base + NCH * ECH
    pltpu.sync_copy(src_hbm.at[pl.ds(toff, TAIL)], src_t)
    pltpu.sync_copy(dst_hbm.at[pl.ds(toff, TAIL)], dst_t)
    pltpu.async_copy(u2_hbm.at[dst_t], u_t, sem_u0)
    pltpu.async_copy(v2_hbm.at[src_t], v_t, sem_v0)

    def tstep(j, _):
        sl = pl.ds(j * 16, 16)
        adj_t[sl] = src_t[sl] - coff
        return 0

    lax.fori_loop(0, TAIL // 16, tstep, 0, unroll=True)
    pltpu.make_async_copy(u2_hbm.at[dst_t], u_t, sem_u0).wait()
    pltpu.make_async_copy(v2_hbm.at[src_t], v_t, sem_v0).wait()
    _edge_chunk_compute(adj_t, u_t, v_t, a_sh, TAIL)

    plsc.subcore_barrier()

    @pl.when(s == 0)
    def _():
        pltpu.sync_copy(a_sh, a_out.at[c])


def _edge(src, dst, u2, v2, zerosa):
    mesh = plsc.VectorSubcoreMesh(core_axis_name="c", subcore_axis_name="s")
    f = pl.kernel(
        _edge_body,
        out_type=jax.ShapeDtypeStruct((2, N // 2, C), jnp.float32),
        mesh=mesh,
        scratch_types=[
            pltpu.VMEM_SHARED((N // 2, C), jnp.float32),
            pltpu.VMEM((ECH,), jnp.int32),
            pltpu.VMEM((ECH,), jnp.int32),
            pltpu.VMEM((ECH,), jnp.int32),
            pltpu.VMEM((ECH,), jnp.int32),
            pltpu.VMEM((ECH,), jnp.int32),
            pltpu.VMEM((ECH,), jnp.int32),
            pltpu.VMEM((ECH, C), jnp.float32),
            pltpu.VMEM((ECH, C), jnp.float32),
            pltpu.VMEM((ECH, C), jnp.float32),
            pltpu.VMEM((ECH, C), jnp.float32),
            pltpu.VMEM((TAIL,), jnp.int32),
            pltpu.VMEM((TAIL,), jnp.int32),
            pltpu.VMEM((TAIL,), jnp.int32),
            pltpu.VMEM((TAIL, C), jnp.float32),
            pltpu.VMEM((TAIL, C), jnp.float32),
            pltpu.SemaphoreType.DMA,
            pltpu.SemaphoreType.DMA,
            pltpu.SemaphoreType.DMA,
            pltpu.SemaphoreType.DMA,
            pltpu.SemaphoreType.DMA,
            pltpu.SemaphoreType.DMA,
        ],
        compiler_params=pltpu.CompilerParams(needs_layout_passes=False),
    )
    return f(src, dst, u2, v2, zerosa)


def _w_body(hist_ref, w_ref):
    cnt = jnp.sum(hist_ref[...], axis=0)
    w = 1.0 / jnp.maximum(cnt, 1.0)
    w_ref[...] = w.reshape(16, 128)


def _w_from_hist(hist_p):
    w2 = pl.pallas_call(
        _w_body,
        grid=(32,),
        in_specs=[pl.BlockSpec((NTILES, 2048), lambda i: (0, i))],
        out_specs=pl.BlockSpec((16, 128), lambda i: (i, 0)),
        out_shape=jax.ShapeDtypeStruct((512, 128), jnp.float32),
    )(hist_p)
    return w2.reshape(B * M, 1)


def _u2_body(lat8_ref, k8_ref, b_ref, wcol_ref, o_ref):
    u = jnp.dot(lat8_ref[...], k8_ref[...],
                preferred_element_type=jnp.float32) + b_ref[...]
    wrep = jnp.broadcast_to(wcol_ref[...], (wcol_ref.shape[0], 16))
    pad = jnp.zeros((u.shape[0], C - KH - 16), jnp.float32)
    o_ref[...] = jnp.concatenate([u, wrep, pad], axis=1)


def _build_u2(lat8, kw1_8, kb1, wcol):
    rows = 1024
    return pl.pallas_call(
        _u2_body,
        grid=(2, M // rows),
        in_specs=[
            pl.BlockSpec((rows, 8), lambda i, j: (j, 0)),
            pl.BlockSpec((8, KH), lambda i, j: (0, 0)),
            pl.BlockSpec((1, KH), lambda i, j: (0, 0)),
            pl.BlockSpec((rows, 1), lambda i, j: (i * (M // rows) + j, 0)),
        ],
        out_specs=pl.BlockSpec((rows, C), lambda i, j: (i * (M // rows) + j, 0)),
        out_shape=jax.ShapeDtypeStruct((B * M, C), jnp.float32),
    )(lat8, kw1_8, kb1, wcol)


def _v2_body(pos8_ref, k8_ref, o_ref):
    v = -jnp.dot(pos8_ref[...], k8_ref[...], preferred_element_type=jnp.float32)
    pad = jnp.zeros((v.shape[0], C - KH), jnp.float32)
    o_ref[...] = jnp.concatenate([v, pad], axis=1)


def _build_v2(pos8, kw1_8):
    return pl.pallas_call(
        _v2_body,
        grid=(N // ROWS,),
        in_specs=[
            pl.BlockSpec((ROWS, 8), lambda i: (i, 0)),
            pl.BlockSpec((8, KH), lambda i: (0, 0)),
        ],
        out_specs=pl.BlockSpec((ROWS, C), lambda i: (i, 0)),
        out_shape=jax.ShapeDtypeStruct((N, C), jnp.float32),
    )(pos8, kw1_8)


def _combine_body(x_ref, a_ref, bidx_ref, wl_ref, bl_ref, kw2_ref,
                  kb2_ref, y_ref, acc_ref):
    i = pl.program_id(0)

    @pl.when(i == 0)
    def _():
        acc_ref[...] = jnp.zeros_like(acc_ref)

    lifted = jnp.dot(x_ref[...], wl_ref[...],
                     preferred_element_type=jnp.float32) + bl_ref[...]
    a = a_ref[...]
    q = jnp.dot(a[:, :KH], kw2_ref[...], preferred_element_type=jnp.float32)
    cw = a[:, KH:KH + 1]
    r = q + cw * kb2_ref[...]
    prod = r * lifted
    bidx = bidx_ref[...]
    for b in range(B):
        mask = (bidx == b).astype(jnp.float32)
        acc_ref[b, :] += jnp.sum(prod * mask, axis=0)

    @pl.when(i == pl.num_programs(0) - 1)
    def _():
        y_ref[...] = acc_ref[...]


def _expand_body(y_ref, wout_ref, bout_ref, bidx_ref, out_ref):
    pooled = y_ref[...] * (1.0 / M)
    rows = jnp.dot(pooled, wout_ref[...],
                   preferred_element_type=jnp.float32) + bout_ref[...]
    bidx = bidx_ref[...]
    out_ref[...] = jnp.where(bidx == 0, rows[0, :][None, :], rows[1, :][None, :])


def kernel(x, pos, W_lift, b_lift, K_W1, K_b1, K_W2, K_b2, W_out, b_out,
           latent_tokens, edge_index, batch_idx):
    dst = edge_index[1]
    lat8 = jnp.pad(latent_tokens, ((0, 0), (0, 5)))
    pos8 = jnp.pad(pos, ((0, 0), (0, 5)))
    kw1_8 = jnp.pad(K_W1, ((0, 5), (0, 0)))
    zeros1 = jnp.zeros((B * M,), jnp.float32)
    zerosa = jnp.zeros((N // 2, C), jnp.float32)

    hist_p = _hist(dst, zeros1)
    wcol = _w_from_hist(hist_p)
    u2 = _build_u2(lat8, kw1_8, K_b1.reshape(1, KH), wcol)
    v2 = _build_v2(pos8, kw1_8)
    a_p = _edge(edge_index[0], dst, u2, v2, zerosa)
    a_full = a_p.reshape(N, C)

    bidx2d = batch_idx.reshape(N, 1)
    grid = N // ROWS
    y = pl.pallas_call(
        _combine_body,
        grid=(grid,),
        in_specs=[
            pl.BlockSpec((ROWS, 128), lambda i: (i, 0)),
            pl.BlockSpec((ROWS, C), lambda i: (i, 0)),
            pl.BlockSpec((ROWS, 1), lambda i: (i, 0)),
            pl.BlockSpec((128, C), lambda i: (0, 0)),
            pl.BlockSpec((1, C), lambda i: (0, 0)),
            pl.BlockSpec((KH, C), lambda i: (0, 0)),
            pl.BlockSpec((1, C), lambda i: (0, 0)),
        ],
        out_specs=pl.BlockSpec((B, C), lambda i: (0, 0)),
        out_shape=jax.ShapeDtypeStruct((B, C), jnp.float32),
        scratch_shapes=[pltpu.VMEM((B, C), jnp.float32)],
    )(x, a_full, bidx2d, W_lift, b_lift.reshape(1, C), K_W2, K_b2.reshape(1, C))

    out = pl.pallas_call(
        _expand_body,
        grid=(grid,),
        in_specs=[
            pl.BlockSpec((B, C), lambda i: (0, 0)),
            pl.BlockSpec((C, OUT), lambda i: (0, 0)),
            pl.BlockSpec((1, OUT), lambda i: (0, 0)),
            pl.BlockSpec((ROWS, 1), lambda i: (i, 0)),
        ],
        out_specs=pl.BlockSpec((ROWS, OUT), lambda i: (i, 0)),
        out_shape=jax.ShapeDtypeStruct((N, OUT), jnp.float32),
    )(y, W_out, b_out.reshape(1, OUT), bidx2d)
    return out


# speedup vs baseline: 2.5455x; 1.0701x over previous
import functools

import jax
import jax.numpy as jnp
from jax import lax
from jax.experimental import pallas as pl
from jax.experimental.pallas import tpu as pltpu
from jax.experimental.pallas import tpu_sc as plsc

N = 10000
B = 2
M = 32 * 32 * 32
KH = 64
C = 128
OUT = 128
E = 320000
NTILES = 32
EPT = E // NTILES
ROWS = 400

HCH = 400


def _hist_body(dst_hbm, zeros_hbm, hist_out, dst0, dst1, hist_v,
               sem_h0, sem_h1):
    c = lax.axis_index("c")
    s = lax.axis_index("s")
    wid = c * 16 + s
    base = wid * EPT
    dstb = (dst0, dst1)
    sem = (sem_h0, sem_h1)

    def issue(i, b):
        pltpu.async_copy(dst_hbm.at[pl.ds(base + i * HCH, HCH)], dstb[b],
                         sem[b])

    def wait(b):
        pltpu.make_async_copy(dst_hbm.at[pl.ds(0, HCH)], dstb[b],
                              sem[b]).wait()

    issue(0, 0)
    issue(1, 1)
    pltpu.sync_copy(zeros_hbm, hist_v)
    ones = jnp.full((16,), 1.0, jnp.float32)
    nch = EPT // HCH

    def chunk(i2, _):
        for b in range(2):
            i = 2 * i2 + b
            wait(b)

            def vstep(j, _):
                d = dstb[b][pl.ds(j * 16, 16)]
                plsc.addupdate_scatter(hist_v, [d], ones)
                return 0

            lax.fori_loop(0, HCH // 16, vstep, 0, unroll=True)

            @pl.when(i + 2 <= nch - 1)
            def _():
                issue(i + 2, b)
        return 0

    lax.fori_loop(0, nch // 2, chunk, 0)
    wait(0)

    def vstep_last(j, _):
        d = dst0[pl.ds(j * 16, 16)]
        plsc.addupdate_scatter(hist_v, [d], ones)
        return 0

    lax.fori_loop(0, HCH // 16, vstep_last, 0, unroll=True)
    pltpu.sync_copy(hist_v, hist_out.at[wid])


def _hist(dst, zeros1):
    mesh = plsc.VectorSubcoreMesh(core_axis_name="c", subcore_axis_name="s")
    f = pl.kernel(
        _hist_body,
        out_type=jax.ShapeDtypeStruct((NTILES, B * M), jnp.float32),
        mesh=mesh,
        scratch_types=[
            pltpu.VMEM((HCH,), jnp.int32),
            pltpu.VMEM((HCH,), jnp.int32),
            pltpu.VMEM((B * M,), jnp.float32),
            pltpu.SemaphoreType.DMA,
            pltpu.SemaphoreType.DMA,
        ],
        compiler_params=pltpu.CompilerParams(needs_layout_passes=False),
    )
    return f(dst, zeros1)


UW = C
VW = C
ECH = 96
NCH = EPT // ECH
TAIL = EPT - NCH * ECH
_G2 = 1.5957691216057308
_GA = 0.044715


def _edge_chunk_compute(adj_b, u_rows, v_rows, a_sh, n_rows):
    def row(r, _):
        wv = u_rows[r, pl.ds(KH, 16)]
        for c4 in range(4):
            sl = pl.ds(c4 * 16, 16)
            z = u_rows[r, sl] + v_rows[r, sl]
            z2 = z * z
            y = _G2 * (z + _GA * z2 * z)
            g = z / (1.0 + jnp.exp(-y))
            u_rows[r, sl] = g * wv
        return 0

    lax.fori_loop(0, n_rows, row, 0)
    pltpu.sync_copy(u_rows, a_sh.at[adj_b], add=True)


def _edge_body(src_hbm, dst_hbm, u2_hbm, v2_hbm, zerosa_hbm,
               a_out,
               a_sh,
               src0, src1, dst0, dst1, adj0, adj1, u0, u1, v0, v1,
               src_t, dst_t, adj_t, u_t, v_t,
               sem_i0, sem_i1, sem_u0, sem_u1, sem_v0, sem_v1):
    c = lax.axis_index("c")
    s = lax.axis_index("s")
    wid = c * 16 + s
    base = wid * EPT
    coff = c * (N // 2)

    @pl.when(s < 5)
    def _():
        pltpu.sync_copy(zerosa_hbm.at[pl.ds(s * 1000, 1000)],
                        a_sh.at[pl.ds(s * 1000, 1000)])

    plsc.subcore_barrier()

    srcb = (src0, src1)
    dstb = (dst0, dst1)
    adjb = (adj0, adj1)
    ur = (u0, u1)
    vr = (v0, v1)

    def adj(b, n16):
        def vstep(j, _):
            sl = pl.ds(j * 16, 16)
            adjb[b][sl] = srcb[b][sl] - coff
            return 0
        lax.fori_loop(0, n16, vstep, 0, unroll=True)
    sem_i = (sem_i0, sem_i1)
    sem_u = (sem_u0, sem_u1)
    sem_v = (sem_v0, sem_v1)
    def issue_idx(ch, b):
        off = base + ch * ECH
        pltpu.async_copy(src_hbm.at[pl.ds(off, ECH)], srcb[b], sem_i[b])
        pltpu.async_copy(dst_hbm.at[pl.ds(off, ECH)], dstb[b], sem_i[b])

    def wait_idx(b):
        pltpu.make_async_copy(src_hbm.at[pl.ds(0, ECH)], srcb[b],
                              sem_i[b]).wait()
        pltpu.make_async_copy(dst_hbm.at[pl.ds(0, ECH)], dstb[b],
                              sem_i[b]).wait()

    def issue_gathers(b):
        pltpu.async_copy(u2_hbm.at[dstb[b]], ur[b], sem_u[b])
        pltpu.async_copy(v2_hbm.at[srcb[b]], vr[b], sem_v[b])

    def wait_gathers(b):
        pltpu.make_async_copy(u2_hbm.at[dstb[b]], ur[b], sem_u[b]).wait()
        pltpu.make_async_copy(v2_hbm.at[srcb[b]], vr[b], sem_v[b]).wait()

    pltpu.sync_copy(src_hbm.at[pl.ds(base, ECH)], srcb[0])
    pltpu.sync_copy(dst_hbm.at[pl.ds(base, ECH)], dstb[0])
    issue_gathers(0)
    adj(0, ECH // 16)
    issue_idx(1, 1)

    def pair(j, _):
        for b in range(2):
            ch = 2 * j + b
            nb = 1 - b
            @pl.when(ch + 1 <= NCH - 1)
            def _():
                wait_idx(nb)
                issue_gathers(nb)
                adj(nb, ECH // 16)

            wait_gathers(b)
            _edge_chunk_compute(adjb[b], ur[b], vr[b], a_sh, ECH)

            @pl.when(ch + 2 <= NCH - 1)
            def _():
                issue_idx(ch + 2, b)
        return 0

    lax.fori_loop(0, NCH // 2, pair, 0)
    toff = base + NCH * ECH
    pltpu.sync_copy(src_hbm.at[pl.ds(toff, TAIL)], src_t)
    pltpu.sync_copy(dst_hbm.at[pl.ds(toff, TAIL)], dst_t)
    pltpu.async_copy(u2_hbm.at[dst_t], u_t, sem_u0)
    pltpu.async_copy(v2_hbm.at[src_t], v_t, sem_v0)

    def tstep(j, _):
        sl = pl.ds(j * 16, 16)
        adj_t[sl] = src_t[sl] - coff
        return 0

    lax.fori_loop(0, TAIL // 16, tstep, 0, unroll=True)
    pltpu.make_async_copy(u2_hbm.at[dst_t], u_t, sem_u0).wait()
    pltpu.make_async_copy(v2_hbm.at[src_t], v_t, sem_v0).wait()
    _edge_chunk_compute(adj_t, u_t, v_t, a_sh, TAIL)

    plsc.subcore_barrier()

    @pl.when(s == 0)
    def _():
        pltpu.sync_copy(a_sh, a_out.at[c])


def _edge(src, dst, u2, v2, zerosa):
    mesh = plsc.VectorSubcoreMesh(core_axis_name="c", subcore_axis_name="s")
    f = pl.kernel(
        _edge_body,
        out_type=jax.ShapeDtypeStruct((2, N // 2, UW), jnp.float32),
        mesh=mesh,
        scratch_types=[
            pltpu.VMEM_SHARED((N // 2, UW), jnp.float32),
            pltpu.VMEM((ECH,), jnp.int32),
            pltpu.VMEM((ECH,), jnp.int32),
            pltpu.VMEM((ECH,), jnp.int32),
            pltpu.VMEM((ECH,), jnp.int32),
            pltpu.VMEM((ECH,), jnp.int32),
            pltpu.VMEM((ECH,), jnp.int32),
            pltpu.VMEM((ECH, UW), jnp.float32),
            pltpu.VMEM((ECH, UW), jnp.float32),
            pltpu.VMEM((ECH, VW), jnp.float32),
            pltpu.VMEM((ECH, VW), jnp.float32),
            pltpu.VMEM((TAIL,), jnp.int32),
            pltpu.VMEM((TAIL,), jnp.int32),
            pltpu.VMEM((TAIL,), jnp.int32),
            pltpu.VMEM((TAIL, UW), jnp.float32),
            pltpu.VMEM((TAIL, VW), jnp.float32),
            pltpu.SemaphoreType.DMA,
            pltpu.SemaphoreType.DMA,
            pltpu.SemaphoreType.DMA,
            pltpu.SemaphoreType.DMA,
            pltpu.SemaphoreType.DMA,
            pltpu.SemaphoreType.DMA,
        ],
        compiler_params=pltpu.CompilerParams(needs_layout_passes=False),
    )
    return f(src, dst, u2, v2, zerosa)


def _u2_body(lat8_ref, k8_ref, b_ref, hist_ref, o_ref):
    u = jnp.dot(lat8_ref[...], k8_ref[...],
                preferred_element_type=jnp.float32) + b_ref[...]
    cnt = jnp.sum(hist_ref[...], axis=0)
    w = 1.0 / jnp.maximum(cnt, 1.0)
    wrep = jnp.broadcast_to(w.reshape(u.shape[0], 1), (u.shape[0], 16))
    pad = jnp.zeros((u.shape[0], UW - KH - 16), jnp.float32)
    o_ref[...] = jnp.concatenate([u, wrep, pad], axis=1)


def _build_u2(lat8, kw1_8, kb1, hist_p):
    rows = 1024
    return pl.pallas_call(
        _u2_body,
        grid=(2, M // rows),
        in_specs=[
            pl.BlockSpec((rows, 8), lambda i, j: (j, 0)),
            pl.BlockSpec((8, KH), lambda i, j: (0, 0)),
            pl.BlockSpec((1, KH), lambda i, j: (0, 0)),
            pl.BlockSpec((NTILES, rows),
                         lambda i, j: (0, i * (M // rows) + j)),
        ],
        out_specs=pl.BlockSpec((rows, UW), lambda i, j: (i * (M // rows) + j, 0)),
        out_shape=jax.ShapeDtypeStruct((B * M, UW), jnp.float32),
    )(lat8, kw1_8, kb1, hist_p)


def _v2_body(pos8_ref, k8_ref, o_ref):
    v = -jnp.dot(pos8_ref[...], k8_ref[...], preferred_element_type=jnp.float32)
    pad = jnp.zeros((v.shape[0], VW - KH), jnp.float32)
    o_ref[...] = jnp.concatenate([v, pad], axis=1)


def _build_v2(pos8, kw1_8):
    return pl.pallas_call(
        _v2_body,
        grid=(N // ROWS,),
        in_specs=[
            pl.BlockSpec((ROWS, 8), lambda i: (i, 0)),
            pl.BlockSpec((8, KH), lambda i: (0, 0)),
        ],
        out_specs=pl.BlockSpec((ROWS, VW), lambda i: (i, 0)),
        out_shape=jax.ShapeDtypeStruct((N, VW), jnp.float32),
    )(pos8, kw1_8)


def _combine_body(x_ref, a_ref, bidx_ref, wl_ref, bl_ref, kw2_ref,
                  kb2_ref, y_ref, acc_ref):
    i = pl.program_id(0)

    @pl.when(i == 0)
    def _():
        acc_ref[...] = jnp.zeros_like(acc_ref)

    lifted = jnp.dot(x_ref[...], wl_ref[...],
                     preferred_element_type=jnp.float32) + bl_ref[...]
    a = a_ref[...]
    q = jnp.dot(a[:, :KH], kw2_ref[...], preferred_element_type=jnp.float32)
    cw = a[:, KH:KH + 1]
    r = q + cw * kb2_ref[...]
    prod = r * lifted
    bidx = bidx_ref[...]
    for b in range(B):
        mask = (bidx == b).astype(jnp.float32)
        acc_ref[b, :] += jnp.sum(prod * mask, axis=0)

    @pl.when(i == pl.num_programs(0) - 1)
    def _():
        y_ref[...] = acc_ref[...]


def _expand_body(y_ref, wout_ref, bout_ref, bidx_ref, out_ref):
    pooled = y_ref[...] * (1.0 / M)
    rows = jnp.dot(pooled, wout_ref[...],
                   preferred_element_type=jnp.float32) + bout_ref[...]
    bidx = bidx_ref[...]
    out_ref[...] = jnp.where(bidx == 0, rows[0, :][None, :], rows[1, :][None, :])


def kernel(x, pos, W_lift, b_lift, K_W1, K_b1, K_W2, K_b2, W_out, b_out,
           latent_tokens, edge_index, batch_idx):
    dst = edge_index[1]
    lat8 = jnp.pad(latent_tokens, ((0, 0), (0, 5)))
    pos8 = jnp.pad(pos, ((0, 0), (0, 5)))
    kw1_8 = jnp.pad(K_W1, ((0, 5), (0, 0)))
    zeros1 = jnp.zeros((B * M,), jnp.float32)
    zerosa = jnp.zeros((N // 2, UW), jnp.float32)

    hist_p = _hist(dst, zeros1)
    u2 = _build_u2(lat8, kw1_8, K_b1.reshape(1, KH), hist_p)
    v2 = _build_v2(pos8, kw1_8)
    a_p = _edge(edge_index[0], dst, u2, v2, zerosa)
    a_full = a_p.reshape(N, UW)

    bidx2d = batch_idx.reshape(N, 1)
    grid = N // ROWS
    y = pl.pallas_call(
        _combine_body,
        grid=(grid,),
        in_specs=[
            pl.BlockSpec((ROWS, 128), lambda i: (i, 0)),
            pl.BlockSpec((ROWS, UW), lambda i: (i, 0)),
            pl.BlockSpec((ROWS, 1), lambda i: (i, 0)),
            pl.BlockSpec((128, C), lambda i: (0, 0)),
            pl.BlockSpec((1, C), lambda i: (0, 0)),
            pl.BlockSpec((KH, C), lambda i: (0, 0)),
            pl.BlockSpec((1, C), lambda i: (0, 0)),
        ],
        out_specs=pl.BlockSpec((B, C), lambda i: (0, 0)),
        out_shape=jax.ShapeDtypeStruct((B, C), jnp.float32),
        scratch_shapes=[pltpu.VMEM((B, C), jnp.float32)],
    )(x, a_full, bidx2d, W_lift, b_lift.reshape(1, C), K_W2, K_b2.reshape(1, C))

    out = pl.pallas_call(
        _expand_body,
        grid=(grid,),
        in_specs=[
            pl.BlockSpec((B, C), lambda i: (0, 0)),
            pl.BlockSpec((C, OUT), lambda i: (0, 0)),
            pl.BlockSpec((1, OUT), lambda i: (0, 0)),
            pl.BlockSpec((ROWS, 1), lambda i: (i, 0)),
        ],
        out_specs=pl.BlockSpec((ROWS, OUT), lambda i: (i, 0)),
        out_shape=jax.ShapeDtypeStruct((N, OUT), jnp.float32),
    )(y, W_out, b_out.reshape(1, OUT), bidx2d)
    return out
